# baseline (speedup 1.0000x reference)
import functools

import jax
import jax.numpy as jnp
from jax import lax
from jax.experimental import pallas as pl
from jax.experimental.pallas import tpu as pltpu
from jax.experimental.pallas import tpu_sc as plsc

_CORES = 2
_SUBS = 16
_WRK = _CORES * _SUBS
_CH = 128
_BLK = 1024


def _round_up(a, m):
    return (a + m - 1) // m * m


def _sc_degree(dstr, zeros, ones):
    npad, d = zeros.shape
    nj = dstr.shape[1]
    rows = npad // _SUBS
    mesh = plsc.VectorSubcoreMesh(core_axis_name="c", subcore_axis_name="s")

    @functools.partial(
        pl.kernel,
        out_type=jax.ShapeDtypeStruct((_CORES, npad, d), jnp.float32),
        mesh=mesh,
        scratch_types=[
            pltpu.VMEM((nj, _CH), jnp.int32),
            pltpu.VMEM((_CH, d), jnp.float32),
            pltpu.VMEM_SHARED((npad, d), jnp.float32),
            pltpu.SemaphoreType.DMA,
        ],
    )
    def k(dst_hbm, zer_hbm, one_hbm, out_hbm, didx, ones_v, acc, sem):
        c = lax.axis_index("c")
        s = lax.axis_index("s")
        w = c * _SUBS + s
        base = s * rows
        pltpu.sync_copy(zer_hbm.at[pl.ds(base, rows)], acc.at[pl.ds(base, rows)])
        pltpu.sync_copy(dst_hbm.at[w], didx)
        pltpu.sync_copy(one_hbm, ones_v)
        plsc.subcore_barrier()

        def fire(j, carry):
            pltpu.async_copy(ones_v, acc.at[didx.at[j]], sem, add=True)
            return carry

        lax.fori_loop(0, nj, fire, 0)

        def drain(j, carry):
            pltpu.make_async_copy(ones_v, acc.at[didx.at[j]], sem).wait()
            return carry

        lax.fori_loop(0, nj, drain, 0)
        plsc.subcore_barrier()
        pltpu.sync_copy(acc.at[pl.ds(base, rows)],
                        out_hbm.at[c, pl.ds(base, rows)])

    return k(dstr, zeros, ones)


def _sc_scatter(y, sdr, zeros):
    npad, d = y.shape
    njh = sdr.shape[3] - 2
    assert njh % 2 == 0
    rows = npad // _SUBS
    mesh = plsc.VectorSubcoreMesh(core_axis_name="c", subcore_axis_name="s")

    @functools.partial(
        pl.kernel,
        out_type=jax.ShapeDtypeStruct((_CORES, npad, d), jnp.float32),
        mesh=mesh,
        scratch_types=[
            pltpu.VMEM((2, njh + 2, _CH), jnp.int32),
            pltpu.VMEM((2, _CH, d), jnp.float32),
            pltpu.VMEM_SHARED((npad, d), jnp.float32),
            pltpu.SemaphoreType.DMA,
            pltpu.SemaphoreType.DMA,
        ],
    )
    def k(y_hbm, sd_hbm, zer_hbm, out_hbm, sd, rows_v, acc, gsem0, gsem1):
        c = lax.axis_index("c")
        s = lax.axis_index("s")
        w = c * _SUBS + s
        base = s * rows
        gsems = (gsem0, gsem1)
        pltpu.sync_copy(zer_hbm.at[pl.ds(base, rows)], acc.at[pl.ds(base, rows)])
        plsc.subcore_barrier()

        for p in (0, 1):
            pltpu.sync_copy(sd_hbm.at[w, p], sd)
            for h in (0, 1):
                pltpu.async_copy(y_hbm.at[sd.at[0, h]], rows_v.at[h],
                                 gsems[h])

            def pair(q, carry):
                for h in (0, 1):
                    j = 2 * q + h
                    pltpu.make_async_copy(y_hbm.at[sd.at[0, j]],
                                          rows_v.at[h], gsems[h]).wait()
                    pltpu.sync_copy(rows_v.at[h], acc.at[sd.at[1, j]],
                                    add=True)
                    pltpu.async_copy(y_hbm.at[sd.at[0, j + 2]],
                                     rows_v.at[h], gsems[h])
                return carry

            lax.fori_loop(0, njh // 2, pair, 0)
            for h in (0, 1):
                pltpu.make_async_copy(y_hbm.at[sd.at[0, njh + h]],
                                      rows_v.at[h], gsems[h]).wait()
        plsc.subcore_barrier()
        pltpu.sync_copy(acc.at[pl.ds(base, rows)],
                        out_hbm.at[c, pl.ds(base, rows)])

    return k(y, sdr, zeros)


def _tc_first(xp, d0, d1, w0, n):
    npad, d = xp.shape
    grid = npad // _BLK

    def body(xr, d0r, d1r, wr, yr, dvr):
        i = pl.program_id(0)
        deg = d0r[:, 0:1] + d1r[:, 0:1] + 1.0
        iv = lax.rsqrt(deg)
        rows = i * _BLK + lax.broadcasted_iota(jnp.int32, (_BLK, d), 0)
        ivf = jnp.where(rows < n, jnp.broadcast_to(iv, (_BLK, d)), 0.0)
        xw = jnp.dot(xr[...], wr[...], preferred_element_type=jnp.float32)
        yr[...] = ivf * xw
        dvr[...] = ivf

    return pl.pallas_call(
        body,
        grid=(grid,),
        in_specs=[
            pl.BlockSpec((_BLK, d), lambda i: (i, 0)),
            pl.BlockSpec((_BLK, d), lambda i: (i, 0)),
            pl.BlockSpec((_BLK, d), lambda i: (i, 0)),
            pl.BlockSpec((d, d), lambda i: (0, 0)),
        ],
        out_specs=[
            pl.BlockSpec((_BLK, d), lambda i: (i, 0)),
            pl.BlockSpec((_BLK, d), lambda i: (i, 0)),
        ],
        out_shape=[
            jax.ShapeDtypeStruct((npad, d), jnp.float32),
            jax.ShapeDtypeStruct((npad, d), jnp.float32),
        ],
    )(xp, d0, d1, w0)


def _tc_mid(z0, z1, yprev, dinvf, wl, ss, tt):
    npad, d = yprev.shape
    grid = npad // _BLK

    def body(z0r, z1r, ypr, dvr, wr, ssr, ttr, yr):
        pre = dvr[...] * (z0r[...] + z1r[...] + ypr[...])
        h = pre * ssr[...] + ttr[...]
        h = jnp.maximum(h, 0.0)
        yr[...] = dvr[...] * jnp.dot(h, wr[...],
                                     preferred_element_type=jnp.float32)

    return pl.pallas_call(
        body,
        grid=(grid,),
        in_specs=[
            pl.BlockSpec((_BLK, d), lambda i: (i, 0)),
            pl.BlockSpec((_BLK, d), lambda i: (i, 0)),
            pl.BlockSpec((_BLK, d), lambda i: (i, 0)),
            pl.BlockSpec((_BLK, d), lambda i: (i, 0)),
            pl.BlockSpec((d, d), lambda i: (0, 0)),
            pl.BlockSpec((1, d), lambda i: (0, 0)),
            pl.BlockSpec((1, d), lambda i: (0, 0)),
        ],
        out_specs=pl.BlockSpec((_BLK, d), lambda i: (i, 0)),
        out_shape=jax.ShapeDtypeStruct((npad, d), jnp.float32),
    )(z0, z1, yprev, dinvf, wl, ss, tt)


def _tc_last(z0, z1, yprev, dinvf, ss, tt):
    npad, d = yprev.shape
    grid = npad // _BLK

    def body(z0r, z1r, ypr, dvr, ssr, ttr, hr):
        pre = dvr[...] * (z0r[...] + z1r[...] + ypr[...])
        hr[...] = pre * ssr[...] + ttr[...]

    return pl.pallas_call(
        body,
        grid=(grid,),
        in_specs=[
            pl.BlockSpec((_BLK, d), lambda i: (i, 0)),
            pl.BlockSpec((_BLK, d), lambda i: (i, 0)),
            pl.BlockSpec((_BLK, d), lambda i: (i, 0)),
            pl.BlockSpec((_BLK, d), lambda i: (i, 0)),
            pl.BlockSpec((1, d), lambda i: (0, 0)),
            pl.BlockSpec((1, d), lambda i: (0, 0)),
        ],
        out_specs=pl.BlockSpec((_BLK, d), lambda i: (i, 0)),
        out_shape=jax.ShapeDtypeStruct((npad, d), jnp.float32),
    )(z0, z1, yprev, dinvf, ss, tt)


def _tc_pool(node_rep, batchp, atomn, mlpwp, mlpbp, g, n):
    npad, d = node_rep.shape

    def body(nrr, bat, atr, wr, br, outr, ner):
        for j in range(g):
            less = jnp.sum(jnp.where(bat[...] < j, 1, 0).astype(jnp.int32))
            idx = atr[j] + less
            ner[pl.ds(j, 1), :] = nrr[pl.ds(idx, 1), :]
        ne = ner[...]
        o = jnp.dot(ne, wr[...], preferred_element_type=jnp.float32) + br[...]
        outr[...] = jnp.where(o > 0, o, 0.1 * o)

    return pl.pallas_call(
        body,
        in_specs=[
            pl.BlockSpec(memory_space=pltpu.VMEM),
            pl.BlockSpec(memory_space=pltpu.VMEM),
            pl.BlockSpec(memory_space=pltpu.SMEM),
            pl.BlockSpec(memory_space=pltpu.VMEM),
            pl.BlockSpec(memory_space=pltpu.VMEM),
        ],
        out_specs=[
            pl.BlockSpec(memory_space=pltpu.VMEM),
            pl.BlockSpec(memory_space=pltpu.VMEM),
        ],
        out_shape=[
            jax.ShapeDtypeStruct((g, d), jnp.float32),
            jax.ShapeDtypeStruct((g, d), jnp.float32),
        ],
    )(node_rep, batchp, atomn, mlpwp, mlpbp)


def kernel(x, edge_index, edge_attr, batch, atom_num, W, b, bn_gamma,
           bn_beta, bn_mean, bn_var, mlp_W, mlp_b):
    n, d = x.shape
    e = edge_index.shape[1]
    g = atom_num.shape[0]
    nl = W.shape[0]
    t = mlp_W.shape[1]

    npad = _round_up(n + 1, _BLK * 2)
    nj = _round_up(-(-e // (_WRK * _CH)), 4)
    ep = _WRK * nj * _CH

    njh = nj // 2
    src = edge_index[0].astype(jnp.int32)
    dst = edge_index[1].astype(jnp.int32)
    padv = jnp.full((ep - e,), n, jnp.int32)
    srcw = jnp.concatenate([src, padv]).reshape(_WRK, 2, njh, _CH)
    dstw = jnp.concatenate([dst, padv]).reshape(_WRK, 2, njh, _CH)
    sdr = jnp.stack([srcw, dstw], axis=2)
    sdr = jnp.pad(sdr, ((0, 0), (0, 0), (0, 0), (0, 2), (0, 0)),
                  constant_values=n)
    dstr = dstw.reshape(_WRK, nj, _CH)

    xp = jnp.pad(x, ((0, npad - n), (0, 0)))
    ones = jnp.ones((_CH, d), jnp.float32)
    zeros = jnp.zeros((npad, d), jnp.float32)

    ss = bn_gamma * lax.rsqrt(bn_var + 1e-5)
    tt = (b - bn_mean) * ss + bn_beta

    degp = _sc_degree(dstr, zeros, ones)
    y, dinvf = _tc_first(xp, degp[0], degp[1], W[0], n)
    node_rep = None
    for l in range(1, nl + 1):
        zp = _sc_scatter(y, sdr, zeros)
        if l < nl:
            y = _tc_mid(zp[0], zp[1], y, dinvf, W[l],
                        ss[l - 1][None], tt[l - 1][None])
        else:
            node_rep = _tc_last(zp[0], zp[1], y, dinvf,
                                ss[l - 1][None], tt[l - 1][None])

    batchp = jnp.concatenate(
        [batch.astype(jnp.int32),
         jnp.full((npad - n,), g, jnp.int32)]).reshape(-1, 128)
    mlpwp = jnp.pad(mlp_W, ((0, 0), (0, d - t)))
    mlpbp = jnp.pad(mlp_b, (0, d - t))[None]
    outp, node_emb = _tc_pool(node_rep, batchp, atom_num.astype(jnp.int32),
                              mlpwp, mlpbp, g, n)
    return outp[:, :t], node_emb


# speedup vs baseline: 1.8591x; 1.8591x over previous
import functools

import jax
import jax.numpy as jnp
from jax import lax
from jax.experimental import pallas as pl
from jax.experimental.pallas import tpu as pltpu
from jax.experimental.pallas import tpu_sc as plsc

_CORES = 2
_SUBS = 16
_WRK = _CORES * _SUBS
_CH = 128
_BLK = 1024


def _round_up(a, m):
    return (a + m - 1) // m * m


def _sc_degree(dstr, zeros, ones):
    npad, d = zeros.shape
    nj = dstr.shape[1]
    rows = npad // _SUBS
    mesh = plsc.VectorSubcoreMesh(core_axis_name="c", subcore_axis_name="s")

    @functools.partial(
        pl.kernel,
        out_type=jax.ShapeDtypeStruct((_CORES, npad, d), jnp.float32),
        mesh=mesh,
        scratch_types=[
            pltpu.VMEM((nj, _CH), jnp.int32),
            pltpu.VMEM((_CH, d), jnp.float32),
            pltpu.VMEM_SHARED((npad, d), jnp.float32),
            pltpu.SemaphoreType.DMA,
        ],
    )
    def k(dst_hbm, zer_hbm, one_hbm, out_hbm, didx, ones_v, acc, sem):
        c = lax.axis_index("c")
        s = lax.axis_index("s")
        w = c * _SUBS + s
        base = s * rows
        pltpu.sync_copy(zer_hbm.at[pl.ds(base, rows)], acc.at[pl.ds(base, rows)])
        pltpu.sync_copy(dst_hbm.at[w], didx)
        pltpu.sync_copy(one_hbm, ones_v)
        plsc.subcore_barrier()

        def fire(j, carry):
            pltpu.async_copy(ones_v, acc.at[didx.at[j]], sem, add=True)
            return carry

        lax.fori_loop(0, nj, fire, 0)

        def drain(j, carry):
            pltpu.make_async_copy(ones_v, acc.at[didx.at[j]], sem).wait()
            return carry

        lax.fori_loop(0, nj, drain, 0)
        plsc.subcore_barrier()
        pltpu.sync_copy(acc.at[pl.ds(base, rows)],
                        out_hbm.at[c, pl.ds(base, rows)])

    return k(dstr, zeros, ones)


def _sc_scatter(y, srcr, dstr, zeros):
    npad, d = y.shape
    nj = srcr.shape[1]
    rows = npad // _SUBS
    mesh = plsc.VectorSubcoreMesh(core_axis_name="c", subcore_axis_name="s")

    @functools.partial(
        pl.kernel,
        out_type=jax.ShapeDtypeStruct((_CORES, npad, d), jnp.float32),
        mesh=mesh,
        scratch_types=[
            pltpu.VMEM((nj, _CH), jnp.int32),
            pltpu.VMEM((nj, _CH), jnp.int32),
            pltpu.VMEM((_CH, d), jnp.float32),
            pltpu.VMEM_SHARED((npad, d), jnp.float32),
            pltpu.SemaphoreType.DMA,
        ],
    )
    def k(y_hbm, src_hbm, dst_hbm, zer_hbm, out_hbm, sidx, didx, rows_v, acc,
          sem):
        c = lax.axis_index("c")
        s = lax.axis_index("s")
        w = c * _SUBS + s
        base = s * rows
        pltpu.sync_copy(zer_hbm.at[pl.ds(base, rows)], acc.at[pl.ds(base, rows)])
        pltpu.sync_copy(src_hbm.at[w], sidx)
        pltpu.sync_copy(dst_hbm.at[w], didx)
        plsc.subcore_barrier()

        def body(j, carry):
            pltpu.async_copy(y_hbm.at[sidx.at[j]], rows_v, sem).wait()
            pltpu.sync_copy(rows_v, acc.at[didx.at[j]], add=True)
            return carry

        lax.fori_loop(0, nj, body, 0)
        plsc.subcore_barrier()
        pltpu.sync_copy(acc.at[pl.ds(base, rows)],
                        out_hbm.at[c, pl.ds(base, rows)])

    return k(y, srcr, dstr, zeros)


def _tc_first(xp, d0, d1, w0, n):
    npad, d = xp.shape
    grid = npad // _BLK

    def body(xr, d0r, d1r, wr, yr, dvr):
        i = pl.program_id(0)
        deg = d0r[:, 0:1] + d1r[:, 0:1] + 1.0
        iv = lax.rsqrt(deg)
        rows = i * _BLK + lax.broadcasted_iota(jnp.int32, (_BLK, d), 0)
        ivf = jnp.where(rows < n, jnp.broadcast_to(iv, (_BLK, d)), 0.0)
        xw = jnp.dot(xr[...], wr[...], preferred_element_type=jnp.float32)
        yr[...] = ivf * xw
        dvr[...] = ivf

    return pl.pallas_call(
        body,
        grid=(grid,),
        in_specs=[
            pl.BlockSpec((_BLK, d), lambda i: (i, 0)),
            pl.BlockSpec((_BLK, d), lambda i: (i, 0)),
            pl.BlockSpec((_BLK, d), lambda i: (i, 0)),
            pl.BlockSpec((d, d), lambda i: (0, 0)),
        ],
        out_specs=[
            pl.BlockSpec((_BLK, d), lambda i: (i, 0)),
            pl.BlockSpec((_BLK, d), lambda i: (i, 0)),
        ],
        out_shape=[
            jax.ShapeDtypeStruct((npad, d), jnp.float32),
            jax.ShapeDtypeStruct((npad, d), jnp.float32),
        ],
    )(xp, d0, d1, w0)


def _tc_mid(z0, z1, yprev, dinvf, wl, ss, tt):
    npad, d = yprev.shape
    grid = npad // _BLK

    def body(z0r, z1r, ypr, dvr, wr, ssr, ttr, yr):
        pre = dvr[...] * (z0r[...] + z1r[...] + ypr[...])
        h = pre * ssr[...] + ttr[...]
        h = jnp.maximum(h, 0.0)
        yr[...] = dvr[...] * jnp.dot(h, wr[...],
                                     preferred_element_type=jnp.float32)

    return pl.pallas_call(
        body,
        grid=(grid,),
        in_specs=[
            pl.BlockSpec((_BLK, d), lambda i: (i, 0)),
            pl.BlockSpec((_BLK, d), lambda i: (i, 0)),
            pl.BlockSpec((_BLK, d), lambda i: (i, 0)),
            pl.BlockSpec((_BLK, d), lambda i: (i, 0)),
            pl.BlockSpec((d, d), lambda i: (0, 0)),
            pl.BlockSpec((1, d), lambda i: (0, 0)),
            pl.BlockSpec((1, d), lambda i: (0, 0)),
        ],
        out_specs=pl.BlockSpec((_BLK, d), lambda i: (i, 0)),
        out_shape=jax.ShapeDtypeStruct((npad, d), jnp.float32),
    )(z0, z1, yprev, dinvf, wl, ss, tt)


def _tc_last(z0, z1, yprev, dinvf, ss, tt):
    npad, d = yprev.shape
    grid = npad // _BLK

    def body(z0r, z1r, ypr, dvr, ssr, ttr, hr):
        pre = dvr[...] * (z0r[...] + z1r[...] + ypr[...])
        hr[...] = pre * ssr[...] + ttr[...]

    return pl.pallas_call(
        body,
        grid=(grid,),
        in_specs=[
            pl.BlockSpec((_BLK, d), lambda i: (i, 0)),
            pl.BlockSpec((_BLK, d), lambda i: (i, 0)),
            pl.BlockSpec((_BLK, d), lambda i: (i, 0)),
            pl.BlockSpec((_BLK, d), lambda i: (i, 0)),
            pl.BlockSpec((1, d), lambda i: (0, 0)),
            pl.BlockSpec((1, d), lambda i: (0, 0)),
        ],
        out_specs=pl.BlockSpec((_BLK, d), lambda i: (i, 0)),
        out_shape=jax.ShapeDtypeStruct((npad, d), jnp.float32),
    )(z0, z1, yprev, dinvf, ss, tt)


def _tc_pool(node_rep, batchp, atomn, mlpwp, mlpbp, g, n):
    npad, d = node_rep.shape

    def body(nrr, bat, atr, wr, br, outr, ner):
        for j in range(g):
            less = jnp.sum(jnp.where(bat[...] < j, 1, 0).astype(jnp.int32))
            idx = atr[j] + less
            ner[pl.ds(j, 1), :] = nrr[pl.ds(idx, 1), :]
        ne = ner[...]
        o = jnp.dot(ne, wr[...], preferred_element_type=jnp.float32) + br[...]
        outr[...] = jnp.where(o > 0, o, 0.1 * o)

    return pl.pallas_call(
        body,
        in_specs=[
            pl.BlockSpec(memory_space=pltpu.VMEM),
            pl.BlockSpec(memory_space=pltpu.VMEM),
            pl.BlockSpec(memory_space=pltpu.SMEM),
            pl.BlockSpec(memory_space=pltpu.VMEM),
            pl.BlockSpec(memory_space=pltpu.VMEM),
        ],
        out_specs=[
            pl.BlockSpec(memory_space=pltpu.VMEM),
            pl.BlockSpec(memory_space=pltpu.VMEM),
        ],
        out_shape=[
            jax.ShapeDtypeStruct((g, d), jnp.float32),
            jax.ShapeDtypeStruct((g, d), jnp.float32),
        ],
    )(node_rep, batchp, atomn, mlpwp, mlpbp)


def kernel(x, edge_index, edge_attr, batch, atom_num, W, b, bn_gamma,
           bn_beta, bn_mean, bn_var, mlp_W, mlp_b):
    n, d = x.shape
    e = edge_index.shape[1]
    g = atom_num.shape[0]
    nl = W.shape[0]
    t = mlp_W.shape[1]

    npad = _round_up(n + 1, _BLK * 2)
    nj = _round_up(-(-e // (_WRK * _CH)), 4)
    ep = _WRK * nj * _CH

    src = edge_index[0].astype(jnp.int32)
    dst = edge_index[1].astype(jnp.int32)
    padv = jnp.full((ep - e,), n, jnp.int32)
    srcr = jnp.concatenate([src, padv]).reshape(_WRK, nj, _CH)
    dstr = jnp.concatenate([dst, padv]).reshape(_WRK, nj, _CH)

    xp = jnp.pad(x, ((0, npad - n), (0, 0)))
    ones = jnp.ones((_CH, d), jnp.float32)
    zeros = jnp.zeros((npad, d), jnp.float32)

    ss = bn_gamma * lax.rsqrt(bn_var + 1e-5)
    tt = (b - bn_mean) * ss + bn_beta

    degp = _sc_degree(dstr, zeros, ones)
    y, dinvf = _tc_first(xp, degp[0], degp[1], W[0], n)
    node_rep = None
    for l in range(1, nl + 1):
        zp = _sc_scatter(y, srcr, dstr, zeros)
        if l < nl:
            y = _tc_mid(zp[0], zp[1], y, dinvf, W[l],
                        ss[l - 1][None], tt[l - 1][None])
        else:
            node_rep = _tc_last(zp[0], zp[1], y, dinvf,
                                ss[l - 1][None], tt[l - 1][None])

    batchp = jnp.concatenate(
        [batch.astype(jnp.int32),
         jnp.full((npad - n,), g, jnp.int32)]).reshape(-1, 128)
    mlpwp = jnp.pad(mlp_W, ((0, 0), (0, d - t)))
    mlpbp = jnp.pad(mlp_b, (0, d - t))[None]
    outp, node_emb = _tc_pool(node_rep, batchp, atom_num.astype(jnp.int32),
                              mlpwp, mlpbp, g, n)
    return outp[:, :t], node_emb


# speedup vs baseline: 2.8674x; 1.5424x over previous
import functools

import jax
import jax.numpy as jnp
from jax import lax
from jax.experimental import pallas as pl
from jax.experimental.pallas import tpu as pltpu
from jax.experimental.pallas import tpu_sc as plsc

_CORES = 2
_SUBS = 16
_WRK = _CORES * _SUBS
_CH = 128
_BLK = 1024


def _round_up(a, m):
    return (a + m - 1) // m * m


def _sc_degree(dstr, zeros, ones):
    npad, d = zeros.shape
    nj = dstr.shape[1]
    rows = npad // _SUBS
    mesh = plsc.VectorSubcoreMesh(core_axis_name="c", subcore_axis_name="s")

    @functools.partial(
        pl.kernel,
        out_type=jax.ShapeDtypeStruct((_CORES, npad, d), jnp.float32),
        mesh=mesh,
        scratch_types=[
            pltpu.VMEM((nj, _CH), jnp.int32),
            pltpu.VMEM((_CH, d), jnp.float32),
            pltpu.VMEM_SHARED((npad, d), jnp.float32),
            pltpu.SemaphoreType.DMA,
        ],
    )
    def k(dst_hbm, zer_hbm, one_hbm, out_hbm, didx, ones_v, acc, sem):
        c = lax.axis_index("c")
        s = lax.axis_index("s")
        w = c * _SUBS + s
        base = s * rows
        pltpu.sync_copy(zer_hbm.at[pl.ds(base, rows)], acc.at[pl.ds(base, rows)])
        pltpu.sync_copy(dst_hbm.at[w], didx)
        pltpu.sync_copy(one_hbm, ones_v)
        plsc.subcore_barrier()

        def fire(j, carry):
            pltpu.sync_copy(ones_v, acc.at[didx.at[j]], add=True)
            return carry

        lax.fori_loop(0, nj, fire, 0)
        plsc.subcore_barrier()
        pltpu.sync_copy(acc.at[pl.ds(base, rows)],
                        out_hbm.at[c, pl.ds(base, rows)])

    return k(dstr, zeros, ones)


def _sc_scatter(y, srcr, dstr, zeros):
    npad, d = y.shape
    nj = srcr.shape[1]
    rows = npad // _SUBS
    mesh = plsc.VectorSubcoreMesh(core_axis_name="c", subcore_axis_name="s")

    @functools.partial(
        pl.kernel,
        out_type=jax.ShapeDtypeStruct((_CORES, npad, d), jnp.float32),
        mesh=mesh,
        scratch_types=[
            pltpu.VMEM((nj, _CH), jnp.int32),
            pltpu.VMEM((nj, _CH), jnp.int32),
            pltpu.VMEM((_CH, d), jnp.float32),
            pltpu.VMEM_SHARED((npad, d), jnp.float32),
            pltpu.SemaphoreType.DMA,
        ],
    )
    def k(y_hbm, src_hbm, dst_hbm, zer_hbm, out_hbm, sidx, didx, rows_v, acc,
          sem):
        c = lax.axis_index("c")
        s = lax.axis_index("s")
        w = c * _SUBS + s
        base = s * rows
        pltpu.sync_copy(zer_hbm.at[pl.ds(base, rows)], acc.at[pl.ds(base, rows)])
        pltpu.sync_copy(src_hbm.at[w], sidx)
        pltpu.sync_copy(dst_hbm.at[w], didx)
        plsc.subcore_barrier()

        def body(j, carry):
            pltpu.async_copy(y_hbm.at[sidx.at[j]], rows_v, sem).wait()
            pltpu.sync_copy(rows_v, acc.at[didx.at[j]], add=True)
            return carry

        lax.fori_loop(0, nj, body, 0)
        plsc.subcore_barrier()
        pltpu.sync_copy(acc.at[pl.ds(base, rows)],
                        out_hbm.at[c, pl.ds(base, rows)])

    return k(y, srcr, dstr, zeros)


def _tc_first(xp, d0, d1, w0, n):
    npad, d = xp.shape
    grid = npad // _BLK

    def body(xr, d0r, d1r, wr, yr, dvr):
        i = pl.program_id(0)
        deg = d0r[:, 0:1] + d1r[:, 0:1] + 1.0
        iv = lax.rsqrt(deg)
        rows = i * _BLK + lax.broadcasted_iota(jnp.int32, (_BLK, d), 0)
        ivf = jnp.where(rows < n, jnp.broadcast_to(iv, (_BLK, d)), 0.0)
        xw = jnp.dot(xr[...], wr[...], preferred_element_type=jnp.float32)
        yr[...] = ivf * xw
        dvr[...] = ivf

    return pl.pallas_call(
        body,
        grid=(grid,),
        in_specs=[
            pl.BlockSpec((_BLK, d), lambda i: (i, 0)),
            pl.BlockSpec((_BLK, d), lambda i: (i, 0)),
            pl.BlockSpec((_BLK, d), lambda i: (i, 0)),
            pl.BlockSpec((d, d), lambda i: (0, 0)),
        ],
        out_specs=[
            pl.BlockSpec((_BLK, d), lambda i: (i, 0)),
            pl.BlockSpec((_BLK, d), lambda i: (i, 0)),
        ],
        out_shape=[
            jax.ShapeDtypeStruct((npad, d), jnp.float32),
            jax.ShapeDtypeStruct((npad, d), jnp.float32),
        ],
    )(xp, d0, d1, w0)


def _tc_mid(z0, z1, yprev, dinvf, wl, ss, tt):
    npad, d = yprev.shape
    grid = npad // _BLK

    def body(z0r, z1r, ypr, dvr, wr, ssr, ttr, yr):
        pre = dvr[...] * (z0r[...] + z1r[...] + ypr[...])
        h = pre * ssr[...] + ttr[...]
        h = jnp.maximum(h, 0.0)
        yr[...] = dvr[...] * jnp.dot(h, wr[...],
                                     preferred_element_type=jnp.float32)

    return pl.pallas_call(
        body,
        grid=(grid,),
        in_specs=[
            pl.BlockSpec((_BLK, d), lambda i: (i, 0)),
            pl.BlockSpec((_BLK, d), lambda i: (i, 0)),
            pl.BlockSpec((_BLK, d), lambda i: (i, 0)),
            pl.BlockSpec((_BLK, d), lambda i: (i, 0)),
            pl.BlockSpec((d, d), lambda i: (0, 0)),
            pl.BlockSpec((1, d), lambda i: (0, 0)),
            pl.BlockSpec((1, d), lambda i: (0, 0)),
        ],
        out_specs=pl.BlockSpec((_BLK, d), lambda i: (i, 0)),
        out_shape=jax.ShapeDtypeStruct((npad, d), jnp.float32),
    )(z0, z1, yprev, dinvf, wl, ss, tt)


def _tc_last(z0, z1, yprev, dinvf, ss, tt):
    npad, d = yprev.shape
    grid = npad // _BLK

    def body(z0r, z1r, ypr, dvr, ssr, ttr, hr):
        pre = dvr[...] * (z0r[...] + z1r[...] + ypr[...])
        hr[...] = pre * ssr[...] + ttr[...]

    return pl.pallas_call(
        body,
        grid=(grid,),
        in_specs=[
            pl.BlockSpec((_BLK, d), lambda i: (i, 0)),
            pl.BlockSpec((_BLK, d), lambda i: (i, 0)),
            pl.BlockSpec((_BLK, d), lambda i: (i, 0)),
            pl.BlockSpec((_BLK, d), lambda i: (i, 0)),
            pl.BlockSpec((1, d), lambda i: (0, 0)),
            pl.BlockSpec((1, d), lambda i: (0, 0)),
        ],
        out_specs=pl.BlockSpec((_BLK, d), lambda i: (i, 0)),
        out_shape=jax.ShapeDtypeStruct((npad, d), jnp.float32),
    )(z0, z1, yprev, dinvf, ss, tt)


def _tc_pool(node_rep, batchp, atomn, mlpwp, mlpbp, g, n):
    npad, d = node_rep.shape

    def body(nrr, bat, atr, wr, br, outr, ner):
        for j in range(g):
            less = jnp.sum(jnp.where(bat[...] < j, 1, 0).astype(jnp.int32))
            idx = atr[j] + less
            ner[pl.ds(j, 1), :] = nrr[pl.ds(idx, 1), :]
        ne = ner[...]
        o = jnp.dot(ne, wr[...], preferred_element_type=jnp.float32) + br[...]
        outr[...] = jnp.where(o > 0, o, 0.1 * o)

    return pl.pallas_call(
        body,
        in_specs=[
            pl.BlockSpec(memory_space=pltpu.VMEM),
            pl.BlockSpec(memory_space=pltpu.VMEM),
            pl.BlockSpec(memory_space=pltpu.SMEM),
            pl.BlockSpec(memory_space=pltpu.VMEM),
            pl.BlockSpec(memory_space=pltpu.VMEM),
        ],
        out_specs=[
            pl.BlockSpec(memory_space=pltpu.VMEM),
            pl.BlockSpec(memory_space=pltpu.VMEM),
        ],
        out_shape=[
            jax.ShapeDtypeStruct((g, d), jnp.float32),
            jax.ShapeDtypeStruct((g, d), jnp.float32),
        ],
    )(node_rep, batchp, atomn, mlpwp, mlpbp)


def kernel(x, edge_index, edge_attr, batch, atom_num, W, b, bn_gamma,
           bn_beta, bn_mean, bn_var, mlp_W, mlp_b):
    n, d = x.shape
    e = edge_index.shape[1]
    g = atom_num.shape[0]
    nl = W.shape[0]
    t = mlp_W.shape[1]

    npad = _round_up(n + 1, _BLK * 2)
    nj = -(-e // (_WRK * _CH))
    ep = _WRK * nj * _CH

    src = edge_index[0].astype(jnp.int32)
    dst = edge_index[1].astype(jnp.int32)
    padv = jnp.full((ep - e,), n, jnp.int32)
    srcr = jnp.concatenate([src, padv]).reshape(_WRK, nj, _CH)
    dstr = jnp.concatenate([dst, padv]).reshape(_WRK, nj, _CH)

    xp = jnp.pad(x, ((0, npad - n), (0, 0)))
    ones = jnp.ones((_CH, d), jnp.float32)
    zeros = jnp.zeros((npad, d), jnp.float32)

    ss = bn_gamma * lax.rsqrt(bn_var + 1e-5)
    tt = (b - bn_mean) * ss + bn_beta

    degp = _sc_degree(dstr, zeros, ones)
    y, dinvf = _tc_first(xp, degp[0], degp[1], W[0], n)
    node_rep = None
    for l in range(1, nl + 1):
        zp = _sc_scatter(y, srcr, dstr, zeros)
        if l < nl:
            y = _tc_mid(zp[0], zp[1], y, dinvf, W[l],
                        ss[l - 1][None], tt[l - 1][None])
        else:
            node_rep = _tc_last(zp[0], zp[1], y, dinvf,
                                ss[l - 1][None], tt[l - 1][None])

    batchp = jnp.concatenate(
        [batch.astype(jnp.int32),
         jnp.full((npad - n,), g, jnp.int32)]).reshape(-1, 128)
    mlpwp = jnp.pad(mlp_W, ((0, 0), (0, d - t)))
    mlpbp = jnp.pad(mlp_b, (0, d - t))[None]
    outp, node_emb = _tc_pool(node_rep, batchp, atom_num.astype(jnp.int32),
                              mlpwp, mlpbp, g, n)
    return outp[:, :t], node_emb
